# Initial kernel scaffold; baseline (speedup 1.0000x reference)
import jax
import jax.numpy as jnp
from jax.experimental import pallas as pl


def kernel(x, edge_index, W1, b1, W2, b2):
    raise NotImplementedError("write your pallas kernel here")


# speedup vs baseline: 8.6774x; 8.6774x over previous
import functools

import jax
import jax.numpy as jnp
from jax import lax
from jax.experimental import pallas as pl
from jax.experimental.pallas import tpu as pltpu
from jax.experimental.pallas import tpu_sc as plsc

NC = 2
NS = 16
LANES = 16


def _make_sc_deg(E, N_pad):
    epw = E // (NC * NS)
    K = 200
    assert epw % K == 0 and epw % 8 == 0
    n_chunks = epw // K
    rpt = N_pad // NS
    assert rpt % LANES == 0
    mesh = plsc.VectorSubcoreMesh(core_axis_name="c", subcore_axis_name="s",
                                  num_cores=NC, num_subcores=NS)

    G = 16

    @functools.partial(
        pl.kernel,
        out_type=jax.ShapeDtypeStruct((NC, N_pad, G), jnp.float32),
        mesh=mesh,
        scratch_types=[
            pltpu.VMEM((K,), jnp.int32),
            pltpu.VMEM((K, G), jnp.float32),
            pltpu.VMEM_SHARED((N_pad, G), jnp.float32),
        ],
    )
    def deg_kernel(dst_hbm, ones_hbm, zeros_hbm, out_hbm, dst_v, ones_v, acc):
        c = lax.axis_index("c")
        s = lax.axis_index("s")
        pltpu.sync_copy(ones_hbm, ones_v)
        pltpu.sync_copy(zeros_hbm, acc.at[pl.ds(s * rpt, rpt)])
        plsc.subcore_barrier()
        wid = c * NS + s

        def body(k, _):
            base = pl.multiple_of(wid * epw + k * K, 8)
            pltpu.sync_copy(dst_hbm.at[pl.ds(base, K)], dst_v)
            pltpu.sync_copy(ones_v, acc.at[dst_v], add=True)
            return 0

        lax.fori_loop(0, n_chunks, body, 0)
        plsc.subcore_barrier()
        pltpu.sync_copy(acc.at[pl.ds(s * rpt, rpt)],
                        out_hbm.at[c].at[pl.ds(s * rpt, rpt)])

    return deg_kernel


def _make_sc_agg(E, N_pad, Dh):
    ept = E // NS
    K = 200
    assert ept % K == 0 and ept % 8 == 0
    n_chunks = ept // K
    rpt = N_pad // NS
    assert rpt % 8 == 0
    mesh = plsc.VectorSubcoreMesh(core_axis_name="c", subcore_axis_name="s",
                                  num_cores=NC, num_subcores=NS)

    @functools.partial(
        pl.kernel,
        out_type=jax.ShapeDtypeStruct((NC, N_pad, Dh), jnp.float32),
        mesh=mesh,
        scratch_types=[
            pltpu.VMEM((K,), jnp.int32),
            pltpu.VMEM((K,), jnp.int32),
            pltpu.VMEM((K, Dh), jnp.float32),
            pltpu.VMEM_SHARED((N_pad, Dh), jnp.float32),
        ],
    )
    def agg_kernel(hp_hbm, src_hbm, dst_hbm, out_hbm, src_v, dst_v, rows_v, acc):
        c = lax.axis_index("c")
        s = lax.axis_index("s")
        pltpu.sync_copy(hp_hbm.at[c].at[pl.ds(s * rpt, rpt)],
                        acc.at[pl.ds(s * rpt, rpt)])
        plsc.subcore_barrier()

        def body(k, _):
            base = pl.multiple_of(s * ept + k * K, 8)
            pltpu.sync_copy(src_hbm.at[pl.ds(base, K)], src_v)
            pltpu.sync_copy(dst_hbm.at[pl.ds(base, K)], dst_v)
            pltpu.sync_copy(hp_hbm.at[c].at[src_v], rows_v)
            pltpu.sync_copy(rows_v, acc.at[dst_v], add=True)
            return 0

        lax.fori_loop(0, n_chunks, body, 0)
        plsc.subcore_barrier()
        pltpu.sync_copy(acc.at[pl.ds(s * rpt, rpt)],
                        out_hbm.at[c].at[pl.ds(s * rpt, rpt)])

    return agg_kernel


def _dinv_blk(deg_ref):
    d = deg_ref[...]
    deg = d[:, 0] + d[:, 1]
    return lax.rsqrt(deg)


def _tc1_body(x_ref, w_ref, deg_ref, o_ref):
    dinv = _dinv_blk(deg_ref)
    h = jnp.dot(x_ref[...], w_ref[...], preferred_element_type=jnp.float32)
    o_ref[0] = h * dinv[:, None]


def _tc2_body(a_ref, deg_ref, b_ref, w_ref, o_ref):
    dinv = _dinv_blk(deg_ref)
    agg = jnp.concatenate([a_ref[0], a_ref[1]], axis=1)
    z = agg * dinv[:, None] + b_ref[...][None, :]
    h = jnp.where(z > 0, z, jnp.exp(jnp.minimum(z, 0.0)) - 1.0)
    o_ref[0] = jnp.dot(h, w_ref[...],
                       preferred_element_type=jnp.float32) * dinv[:, None]


def _tc3_body(a_ref, deg_ref, b_ref, o_ref):
    dinv = _dinv_blk(deg_ref)
    agg = jnp.concatenate([a_ref[0], a_ref[1]], axis=1)
    o_ref[...] = agg * dinv[:, None] + b_ref[...][None, :]


def _tc1(x, W, deg2, N_pad, blk):
    N, Din = x.shape
    Dh = W.shape[1] // NC
    grid = (NC, N // blk)
    return pl.pallas_call(
        _tc1_body,
        grid=grid,
        in_specs=[
            pl.BlockSpec((blk, Din), lambda c, i: (i, 0)),
            pl.BlockSpec((Din, Dh), lambda c, i: (0, c)),
            pl.BlockSpec((blk, NC), lambda c, i: (i, 0)),
        ],
        out_specs=pl.BlockSpec((1, blk, Dh), lambda c, i: (c, i, 0)),
        out_shape=jax.ShapeDtypeStruct((NC, N_pad, Dh), jnp.float32),
    )(x, W, deg2)


def _tc2(agg, deg2, b, W, blk):
    _, N_pad, Dh = agg.shape
    N = deg2.shape[0]
    D = W.shape[0]
    grid = (NC, N // blk)
    return pl.pallas_call(
        _tc2_body,
        grid=grid,
        in_specs=[
            pl.BlockSpec((NC, blk, Dh), lambda c, i: (0, i, 0)),
            pl.BlockSpec((blk, NC), lambda c, i: (i, 0)),
            pl.BlockSpec((D,), lambda c, i: (0,)),
            pl.BlockSpec((D, Dh), lambda c, i: (0, c)),
        ],
        out_specs=pl.BlockSpec((1, blk, Dh), lambda c, i: (c, i, 0)),
        out_shape=jax.ShapeDtypeStruct((NC, N_pad, Dh), jnp.float32),
    )(agg, deg2, b, W)


def _tc3(agg, deg2, b, blk):
    _, N_pad, Dh = agg.shape
    N = deg2.shape[0]
    D = Dh * NC
    grid = (N // blk,)
    return pl.pallas_call(
        _tc3_body,
        grid=grid,
        in_specs=[
            pl.BlockSpec((NC, blk, Dh), lambda i: (0, i, 0)),
            pl.BlockSpec((blk, NC), lambda i: (i, 0)),
            pl.BlockSpec((D,), lambda i: (0,)),
        ],
        out_specs=pl.BlockSpec((blk, D), lambda i: (i, 0)),
        out_shape=jax.ShapeDtypeStruct((N, D), jnp.float32),
    )(agg, deg2, b)


@jax.jit
def kernel(x, edge_index, W1, b1, W2, b2):
    N, Din = x.shape
    E = edge_index.shape[1]
    Dh = W1.shape[1] // NC
    N_pad = ((N + NS * LANES - 1) // (NS * LANES)) * (NS * LANES)
    blk = 1000

    src = edge_index[0]
    dst = edge_index[1]

    sc_agg = _make_sc_agg(E, N_pad, Dh)

    ones_hp = jnp.ones((NC, N_pad, Dh), jnp.float32)
    dega = sc_agg(ones_hp, src, dst)
    deg2 = jnp.stack([dega[0, :N, 0], jnp.zeros((N,), jnp.float32)], axis=1)

    hp1 = _tc1(x, W1, deg2, N_pad, blk)
    agg1 = sc_agg(hp1, src, dst)
    hp2 = _tc2(agg1, deg2, b1, W2, blk)
    agg2 = sc_agg(hp2, src, dst)
    return _tc3(agg2, deg2, b2, blk)


# speedup vs baseline: 11.8236x; 1.3626x over previous
import functools

import jax
import jax.numpy as jnp
from jax import lax
from jax.experimental import pallas as pl
from jax.experimental.pallas import tpu as pltpu
from jax.experimental.pallas import tpu_sc as plsc

NC = 2
NS = 16
LANES = 16


def _make_sc_deg(E, N_pad):
    epw = E // (NC * NS)
    K = 200
    assert epw % K == 0 and epw % 8 == 0
    n_chunks = epw // K
    rpt = N_pad // NS
    assert rpt % LANES == 0
    mesh = plsc.VectorSubcoreMesh(core_axis_name="c", subcore_axis_name="s",
                                  num_cores=NC, num_subcores=NS)

    G = 16

    @functools.partial(
        pl.kernel,
        out_type=jax.ShapeDtypeStruct((NC, N_pad, G), jnp.float32),
        mesh=mesh,
        scratch_types=[
            pltpu.VMEM((K,), jnp.int32),
            pltpu.VMEM((K, G), jnp.float32),
            pltpu.VMEM_SHARED((N_pad, G), jnp.float32),
        ],
    )
    def deg_kernel(dst_hbm, ones_hbm, zeros_hbm, out_hbm, dst_v, ones_v, acc):
        c = lax.axis_index("c")
        s = lax.axis_index("s")
        pltpu.sync_copy(ones_hbm, ones_v)
        pltpu.sync_copy(zeros_hbm, acc.at[pl.ds(s * rpt, rpt)])
        plsc.subcore_barrier()
        wid = c * NS + s

        def body(k, _):
            base = pl.multiple_of(wid * epw + k * K, 8)
            pltpu.sync_copy(dst_hbm.at[pl.ds(base, K)], dst_v)
            pltpu.sync_copy(ones_v, acc.at[dst_v], add=True)
            return 0

        lax.fori_loop(0, n_chunks, body, 0)
        plsc.subcore_barrier()
        pltpu.sync_copy(acc.at[pl.ds(s * rpt, rpt)],
                        out_hbm.at[c].at[pl.ds(s * rpt, rpt)])

    return deg_kernel


def _make_sc_agg(E_pad, N_pad, Dh):
    ept = E_pad // NS
    K = 160
    assert ept % K == 0 and ept % 8 == 0
    n_chunks = ept // K
    assert n_chunks % 2 == 0
    rpt = N_pad // NS
    assert rpt % 8 == 0
    mesh = plsc.VectorSubcoreMesh(core_axis_name="c", subcore_axis_name="s",
                                  num_cores=NC, num_subcores=NS)

    @functools.partial(
        pl.kernel,
        out_type=jax.ShapeDtypeStruct((NC, N_pad, Dh), jnp.float32),
        mesh=mesh,
        scratch_types=[
            pltpu.VMEM((K,), jnp.int32),
            pltpu.VMEM((K,), jnp.int32),
            pltpu.VMEM((K,), jnp.int32),
            pltpu.VMEM((K,), jnp.int32),
            pltpu.VMEM((K, Dh), jnp.float32),
            pltpu.VMEM((K, Dh), jnp.float32),
            pltpu.SemaphoreType.DMA,
            pltpu.SemaphoreType.DMA,
            pltpu.VMEM_SHARED((N_pad, Dh), jnp.float32),
        ],
    )
    def agg_kernel(hp_hbm, src_hbm, dst_hbm, out_hbm,
                   src_v0, src_v1, dst_v0, dst_v1, rows_v0, rows_v1,
                   sem0, sem1, acc):
        c = lax.axis_index("c")
        s = lax.axis_index("s")
        hp_c = hp_hbm.at[c]
        srcs = (src_v0, src_v1)
        dsts = (dst_v0, dst_v1)
        rows = (rows_v0, rows_v1)
        sems = (sem0, sem1)
        pltpu.sync_copy(hp_c.at[pl.ds(s * rpt, rpt)],
                        acc.at[pl.ds(s * rpt, rpt)])
        plsc.subcore_barrier()

        def start_chunk(k, b):
            base = pl.multiple_of(s * ept + k * K, 8)
            pltpu.sync_copy(src_hbm.at[pl.ds(base, K)], srcs[b])
            pltpu.sync_copy(dst_hbm.at[pl.ds(base, K)], dsts[b])
            pltpu.async_copy(hp_c.at[srcs[b]], rows[b], sems[b])

        def finish_chunk(b):
            pltpu.make_async_copy(hp_c.at[srcs[b]], rows[b], sems[b]).wait()
            pltpu.sync_copy(rows[b], acc.at[dsts[b]], add=True)

        start_chunk(0, 0)

        def body(k2, _):
            k = k2 * 2
            start_chunk(k + 1, 1)
            finish_chunk(0)

            @pl.when(k + 2 < n_chunks)
            def _():
                start_chunk(k + 2, 0)

            finish_chunk(1)
            return 0

        lax.fori_loop(0, n_chunks // 2, body, 0)
        plsc.subcore_barrier()
        pltpu.sync_copy(acc.at[pl.ds(s * rpt, rpt)],
                        out_hbm.at[c].at[pl.ds(s * rpt, rpt)])

    return agg_kernel


def _dinv_blk(deg_ref):
    d = deg_ref[...]
    deg = d[:, 0] + d[:, 1]
    return lax.rsqrt(deg)


def _tc1_body(x_ref, w_ref, deg_ref, o_ref):
    dinv = _dinv_blk(deg_ref)
    h = jnp.dot(x_ref[...], w_ref[...], preferred_element_type=jnp.float32)
    o_ref[0] = h * dinv[:, None]


def _tc2_body(a_ref, deg_ref, b_ref, w_ref, o_ref):
    dinv = _dinv_blk(deg_ref)
    agg = jnp.concatenate([a_ref[0], a_ref[1]], axis=1)
    z = agg * dinv[:, None] + b_ref[...][None, :]
    h = jnp.where(z > 0, z, jnp.exp(jnp.minimum(z, 0.0)) - 1.0)
    o_ref[0] = jnp.dot(h, w_ref[...],
                       preferred_element_type=jnp.float32) * dinv[:, None]


def _tc3_body(a_ref, deg_ref, b_ref, o_ref):
    dinv = _dinv_blk(deg_ref)
    agg = jnp.concatenate([a_ref[0], a_ref[1]], axis=1)
    o_ref[...] = agg * dinv[:, None] + b_ref[...][None, :]


def _tc1(x, W, deg2, N_pad, blk):
    N, Din = x.shape
    Dh = W.shape[1] // NC
    grid = (NC, N // blk)
    return pl.pallas_call(
        _tc1_body,
        grid=grid,
        in_specs=[
            pl.BlockSpec((blk, Din), lambda c, i: (i, 0)),
            pl.BlockSpec((Din, Dh), lambda c, i: (0, c)),
            pl.BlockSpec((blk, NC), lambda c, i: (i, 0)),
        ],
        out_specs=pl.BlockSpec((1, blk, Dh), lambda c, i: (c, i, 0)),
        out_shape=jax.ShapeDtypeStruct((NC, N_pad, Dh), jnp.float32),
    )(x, W, deg2)


def _tc2(agg, deg2, b, W, blk):
    _, N_pad, Dh = agg.shape
    N = deg2.shape[0]
    D = W.shape[0]
    grid = (NC, N // blk)
    return pl.pallas_call(
        _tc2_body,
        grid=grid,
        in_specs=[
            pl.BlockSpec((NC, blk, Dh), lambda c, i: (0, i, 0)),
            pl.BlockSpec((blk, NC), lambda c, i: (i, 0)),
            pl.BlockSpec((D,), lambda c, i: (0,)),
            pl.BlockSpec((D, Dh), lambda c, i: (0, c)),
        ],
        out_specs=pl.BlockSpec((1, blk, Dh), lambda c, i: (c, i, 0)),
        out_shape=jax.ShapeDtypeStruct((NC, N_pad, Dh), jnp.float32),
    )(agg, deg2, b, W)


def _tc3(agg, deg2, b, blk):
    _, N_pad, Dh = agg.shape
    N = deg2.shape[0]
    D = Dh * NC
    grid = (N // blk,)
    return pl.pallas_call(
        _tc3_body,
        grid=grid,
        in_specs=[
            pl.BlockSpec((NC, blk, Dh), lambda i: (0, i, 0)),
            pl.BlockSpec((blk, NC), lambda i: (i, 0)),
            pl.BlockSpec((D,), lambda i: (0,)),
        ],
        out_specs=pl.BlockSpec((blk, D), lambda i: (i, 0)),
        out_shape=jax.ShapeDtypeStruct((N, D), jnp.float32),
    )(agg, deg2, b)


@jax.jit
def kernel(x, edge_index, W1, b1, W2, b2):
    N, Din = x.shape
    E = edge_index.shape[1]
    Dh = W1.shape[1] // NC
    N_pad = ((N + NS * LANES - 1) // (NS * LANES)) * (NS * LANES)
    blk = 1000

    src = edge_index[0]
    dst = edge_index[1]

    gran = NS * 160 * 2
    E_pad = ((E + gran - 1) // gran) * gran
    n_extra = E_pad - E
    pad_idx = N + (jnp.arange(n_extra, dtype=jnp.int32) % (N_pad - N))
    src = jnp.concatenate([src, pad_idx])
    dst = jnp.concatenate([dst, pad_idx])

    sc_agg = _make_sc_agg(E_pad, N_pad, Dh)

    ones_hp = jnp.ones((NC, N_pad, Dh), jnp.float32)
    dega = sc_agg(ones_hp, src, dst)
    deg2 = jnp.stack([dega[0, :N, 0], jnp.zeros((N,), jnp.float32)], axis=1)

    hp1 = _tc1(x, W1, deg2, N_pad, blk)
    agg1 = sc_agg(hp1, src, dst)
    hp2 = _tc2(agg1, deg2, b1, W2, blk)
    agg2 = sc_agg(hp2, src, dst)
    return _tc3(agg2, deg2, b2, blk)


# speedup vs baseline: 14.9836x; 1.2673x over previous
import functools

import jax
import jax.numpy as jnp
from jax import lax
from jax.experimental import pallas as pl
from jax.experimental.pallas import tpu as pltpu
from jax.experimental.pallas import tpu_sc as plsc

NC = 2
NS = 16
LANES = 16


def _make_sc_deg(E_pad, N_pad):
    epw = E_pad // (NC * NS)
    CH = 1280
    assert epw % CH == 0 and CH % LANES == 0
    n_ch = epw // CH
    R = N_pad // 128
    mesh = plsc.VectorSubcoreMesh(core_axis_name="c", subcore_axis_name="s",
                                  num_cores=NC, num_subcores=NS)

    @functools.partial(
        pl.kernel,
        out_type=jax.ShapeDtypeStruct((NC * NS, R, 128), jnp.float32),
        mesh=mesh,
        compiler_params=pltpu.CompilerParams(needs_layout_passes=False),
        scratch_types=[
            pltpu.VMEM((CH,), jnp.int32),
            pltpu.VMEM((R, 128), jnp.float32),
        ],
    )
    def deg_kernel(dst_hbm, zeros_hbm, out_hbm, dst_v, hist):
        c = lax.axis_index("c")
        s = lax.axis_index("s")
        wid = c * NS + s
        pltpu.sync_copy(zeros_hbm, hist)
        ones = jnp.full((LANES,), 1.0, jnp.float32)

        def chunk(k, _):
            base = pl.multiple_of(wid * epw + k * CH, 8)
            pltpu.sync_copy(dst_hbm.at[pl.ds(base, CH)], dst_v)

            def vec(j, _):
                iv = dst_v[pl.ds(j * LANES, LANES)]
                ir = lax.shift_right_logical(iv, 7)
                il = lax.bitwise_and(iv, 127)
                plsc.addupdate_scatter(hist, [ir, il], ones)
                return 0

            lax.fori_loop(0, CH // LANES, vec, 0)
            return 0

        lax.fori_loop(0, n_ch, chunk, 0)
        pltpu.sync_copy(hist, out_hbm.at[wid])

    return deg_kernel


def _make_sc_agg(E_pad, N_pad, Dh):
    ept = E_pad // NS
    K = 128
    assert ept % K == 0 and ept % 8 == 0
    n_chunks = ept // K
    assert n_chunks % 2 == 0
    rpt = N_pad // NS
    assert rpt % 8 == 0
    mesh = plsc.VectorSubcoreMesh(core_axis_name="c", subcore_axis_name="s",
                                  num_cores=NC, num_subcores=NS)

    @functools.partial(
        pl.kernel,
        out_type=jax.ShapeDtypeStruct((NC, N_pad, Dh), jnp.float32),
        mesh=mesh,
        scratch_types=[
            pltpu.VMEM((K,), jnp.int32),
            pltpu.VMEM((K,), jnp.int32),
            pltpu.VMEM((K,), jnp.int32),
            pltpu.VMEM((K,), jnp.int32),
            pltpu.VMEM((K, Dh), jnp.float32),
            pltpu.VMEM((K, Dh), jnp.float32),
            pltpu.SemaphoreType.DMA,
            pltpu.SemaphoreType.DMA,
            pltpu.VMEM_SHARED((N_pad, Dh), jnp.float32),
        ],
    )
    def agg_kernel(hp_hbm, src_hbm, dst_hbm, out_hbm,
                   src_v0, src_v1, dst_v0, dst_v1, rows_v0, rows_v1,
                   sem0, sem1, acc):
        c = lax.axis_index("c")
        s = lax.axis_index("s")
        hp_c = hp_hbm.at[c]
        srcs = (src_v0, src_v1)
        dsts = (dst_v0, dst_v1)
        rows = (rows_v0, rows_v1)
        sems = (sem0, sem1)
        pltpu.sync_copy(hp_c.at[pl.ds(s * rpt, rpt)],
                        acc.at[pl.ds(s * rpt, rpt)])
        plsc.subcore_barrier()

        def start_chunk(k, b):
            base = pl.multiple_of(s * ept + k * K, 8)
            pltpu.sync_copy(src_hbm.at[pl.ds(base, K)], srcs[b])
            pltpu.sync_copy(dst_hbm.at[pl.ds(base, K)], dsts[b])
            pltpu.async_copy(hp_c.at[srcs[b]], rows[b], sems[b])

        def finish_chunk(b):
            pltpu.make_async_copy(hp_c.at[srcs[b]], rows[b], sems[b]).wait()
            pltpu.sync_copy(rows[b], acc.at[dsts[b]], add=True)

        start_chunk(0, 0)

        def body(k2, _):
            k = k2 * 2
            start_chunk(k + 1, 1)
            finish_chunk(0)

            @pl.when(k + 2 < n_chunks)
            def _():
                start_chunk(k + 2, 0)

            finish_chunk(1)
            return 0

        lax.fori_loop(0, n_chunks // 2, body, 0)
        plsc.subcore_barrier()
        pltpu.sync_copy(acc.at[pl.ds(s * rpt, rpt)],
                        out_hbm.at[c].at[pl.ds(s * rpt, rpt)])

    return agg_kernel


def _dinv_blk(deg_ref):
    d = deg_ref[...]
    deg = jnp.sum(d, axis=1) + 1.0
    return lax.rsqrt(deg)


def _tc1_body(x_ref, w_ref, deg_ref, o_ref):
    dinv = _dinv_blk(deg_ref)
    h = jnp.dot(x_ref[...], w_ref[...], preferred_element_type=jnp.float32)
    o_ref[0] = h * dinv[:, None]


def _tc2_body(a_ref, deg_ref, b_ref, w_ref, o_ref):
    dinv = _dinv_blk(deg_ref)
    agg = jnp.concatenate([a_ref[0], a_ref[1]], axis=1)
    z = agg * dinv[:, None] + b_ref[...][None, :]
    h = jnp.where(z > 0, z, jnp.exp(jnp.minimum(z, 0.0)) - 1.0)
    o_ref[0] = jnp.dot(h, w_ref[...],
                       preferred_element_type=jnp.float32) * dinv[:, None]


def _tc3_body(a_ref, deg_ref, b_ref, o_ref):
    dinv = _dinv_blk(deg_ref)
    agg = jnp.concatenate([a_ref[0], a_ref[1]], axis=1)
    o_ref[...] = agg * dinv[:, None] + b_ref[...][None, :]


def _tc1(x, W, deg2, N_pad, blk):
    N, Din = x.shape
    Dh = W.shape[1] // NC
    grid = (NC, N // blk)
    return pl.pallas_call(
        _tc1_body,
        grid=grid,
        in_specs=[
            pl.BlockSpec((blk, Din), lambda c, i: (i, 0)),
            pl.BlockSpec((Din, Dh), lambda c, i: (0, c)),
            pl.BlockSpec((blk, NC * NS), lambda c, i: (i, 0)),
        ],
        out_specs=pl.BlockSpec((1, blk, Dh), lambda c, i: (c, i, 0)),
        out_shape=jax.ShapeDtypeStruct((NC, N_pad, Dh), jnp.float32),
    )(x, W, deg2)


def _tc2(agg, deg2, b, W, blk):
    _, N_pad, Dh = agg.shape
    N = deg2.shape[0]
    D = W.shape[0]
    grid = (NC, N // blk)
    return pl.pallas_call(
        _tc2_body,
        grid=grid,
        in_specs=[
            pl.BlockSpec((NC, blk, Dh), lambda c, i: (0, i, 0)),
            pl.BlockSpec((blk, NC * NS), lambda c, i: (i, 0)),
            pl.BlockSpec((D,), lambda c, i: (0,)),
            pl.BlockSpec((D, Dh), lambda c, i: (0, c)),
        ],
        out_specs=pl.BlockSpec((1, blk, Dh), lambda c, i: (c, i, 0)),
        out_shape=jax.ShapeDtypeStruct((NC, N_pad, Dh), jnp.float32),
    )(agg, deg2, b, W)


def _tc3(agg, deg2, b, blk):
    _, N_pad, Dh = agg.shape
    N = deg2.shape[0]
    D = Dh * NC
    grid = (N // blk,)
    return pl.pallas_call(
        _tc3_body,
        grid=grid,
        in_specs=[
            pl.BlockSpec((NC, blk, Dh), lambda i: (0, i, 0)),
            pl.BlockSpec((blk, NC * NS), lambda i: (i, 0)),
            pl.BlockSpec((D,), lambda i: (0,)),
        ],
        out_specs=pl.BlockSpec((blk, D), lambda i: (i, 0)),
        out_shape=jax.ShapeDtypeStruct((N, D), jnp.float32),
    )(agg, deg2, b)


@jax.jit
def kernel(x, edge_index, W1, b1, W2, b2):
    N, Din = x.shape
    E = edge_index.shape[1]
    Dh = W1.shape[1] // NC
    N_pad = ((N + NS * LANES - 1) // (NS * LANES)) * (NS * LANES)
    blk = 1000

    src = edge_index[0]
    dst = edge_index[1]

    gran = NS * 128 * 2
    E_pad = ((E + gran - 1) // gran) * gran
    n_extra = E_pad - E
    pad_idx = N + (jnp.arange(n_extra, dtype=jnp.int32) % (N_pad - N))
    src = jnp.concatenate([src, pad_idx])
    dst = jnp.concatenate([dst, pad_idx])

    sc_agg = _make_sc_agg(E_pad, N_pad, Dh)

    zeros_h = jnp.zeros((N_pad // 128, 128), jnp.float32)
    degp = _make_sc_deg(E_pad, N_pad)(dst, zeros_h)
    deg2 = degp.reshape(NC * NS, N_pad)[:, :N].T

    hp1 = _tc1(x, W1, deg2, N_pad, blk)
    agg1 = sc_agg(hp1, src, dst)
    hp2 = _tc2(agg1, deg2, b1, W2, blk)
    agg2 = sc_agg(hp2, src, dst)
    return _tc3(agg2, deg2, b2, blk)


# speedup vs baseline: 17.6901x; 1.1806x over previous
import functools

import jax
import jax.numpy as jnp
from jax import lax
from jax.experimental import pallas as pl
from jax.experimental.pallas import tpu as pltpu
from jax.experimental.pallas import tpu_sc as plsc

NC = 2
NS = 16
LANES = 16


def _make_sc_deg(E_pad, N_pad):
    epw = E_pad // (NC * NS)
    CH = 1280
    assert epw % CH == 0 and CH % LANES == 0
    n_ch = epw // CH
    R = N_pad // 128
    mesh = plsc.VectorSubcoreMesh(core_axis_name="c", subcore_axis_name="s",
                                  num_cores=NC, num_subcores=NS)

    @functools.partial(
        pl.kernel,
        out_type=jax.ShapeDtypeStruct((NC * NS, R, 128), jnp.float32),
        mesh=mesh,
        compiler_params=pltpu.CompilerParams(needs_layout_passes=False),
        scratch_types=[
            pltpu.VMEM((CH,), jnp.int32),
            pltpu.VMEM((R, 128), jnp.float32),
        ],
    )
    def deg_kernel(dst_hbm, zeros_hbm, out_hbm, dst_v, hist):
        c = lax.axis_index("c")
        s = lax.axis_index("s")
        wid = c * NS + s
        pltpu.sync_copy(zeros_hbm, hist)
        ones = jnp.full((LANES,), 1.0, jnp.float32)

        def chunk(k, _):
            base = pl.multiple_of(wid * epw + k * CH, 8)
            pltpu.sync_copy(dst_hbm.at[pl.ds(base, CH)], dst_v)

            def vec(j, _):
                iv = dst_v[pl.ds(j * LANES, LANES)]
                ir = lax.shift_right_logical(iv, 7)
                il = lax.bitwise_and(iv, 127)
                plsc.addupdate_scatter(hist, [ir, il], ones)
                return 0

            lax.fori_loop(0, CH // LANES, vec, 0)
            return 0

        lax.fori_loop(0, n_ch, chunk, 0)
        pltpu.sync_copy(hist, out_hbm.at[wid])

    return deg_kernel


def _make_sc_agg(E_pad, N_pad, Dh):
    ept = E_pad // NS
    K = 128
    B = 16
    assert ept % (K * B) == 0
    n_bulks = ept // (K * B)
    rowb = ept // K
    rpt = N_pad // NS
    assert rpt % 8 == 0
    mesh = plsc.VectorSubcoreMesh(core_axis_name="c", subcore_axis_name="s",
                                  num_cores=NC, num_subcores=NS)

    @functools.partial(
        pl.kernel,
        out_type=jax.ShapeDtypeStruct((NC, N_pad, Dh), jnp.float32),
        mesh=mesh,
        scratch_types=[
            pltpu.VMEM((B, K), jnp.int32),
            pltpu.VMEM((B, K), jnp.int32),
            pltpu.VMEM((K, Dh), jnp.float32),
            pltpu.VMEM((K, Dh), jnp.float32),
            pltpu.SemaphoreType.DMA,
            pltpu.SemaphoreType.DMA,
            pltpu.VMEM_SHARED((N_pad, Dh), jnp.float32),
        ],
    )
    def agg_kernel(hp_hbm, src_hbm, dst_hbm, out_hbm,
                   sidx, didx, rows_v0, rows_v1, sem0, sem1, acc):
        c = lax.axis_index("c")
        s = lax.axis_index("s")
        hp_c = hp_hbm.at[c]
        rows = (rows_v0, rows_v1)
        sems = (sem0, sem1)
        pltpu.sync_copy(hp_c.at[pl.ds(s * rpt, rpt)],
                        acc.at[pl.ds(s * rpt, rpt)])
        plsc.subcore_barrier()

        def start_chunk(j, b):
            pltpu.async_copy(hp_c.at[sidx.at[j]], rows[b], sems[b])

        def finish_chunk(j, b):
            pltpu.make_async_copy(hp_c.at[sidx.at[j]], rows[b],
                                  sems[b]).wait()
            pltpu.sync_copy(rows[b], acc.at[didx.at[j]], add=True)

        def bulk(m, _):
            row0 = pl.multiple_of(s * rowb + m * B, 8)
            pltpu.sync_copy(src_hbm.at[pl.ds(row0, B)], sidx)
            pltpu.sync_copy(dst_hbm.at[pl.ds(row0, B)], didx)
            start_chunk(0, 0)

            def pair(j2, _):
                j = j2 * 2
                start_chunk(j + 1, 1)
                finish_chunk(j, 0)

                @pl.when(j + 2 < B)
                def _():
                    start_chunk(j + 2, 0)

                finish_chunk(j + 1, 1)
                return 0

            lax.fori_loop(0, B // 2, pair, 0)
            return 0

        lax.fori_loop(0, n_bulks, bulk, 0)
        plsc.subcore_barrier()
        pltpu.sync_copy(acc.at[pl.ds(s * rpt, rpt)],
                        out_hbm.at[c].at[pl.ds(s * rpt, rpt)])

    return agg_kernel


def _dinv_blk(deg_ref):
    d = deg_ref[...]
    deg = jnp.sum(d, axis=1) + 1.0
    return lax.rsqrt(deg)


def _tc1_body(x_ref, w_ref, deg_ref, o_ref):
    dinv = _dinv_blk(deg_ref)
    h = jnp.dot(x_ref[...], w_ref[...], preferred_element_type=jnp.float32)
    o_ref[0] = h * dinv[:, None]


def _tc2_body(a_ref, deg_ref, b_ref, w_ref, o_ref):
    dinv = _dinv_blk(deg_ref)
    agg = jnp.concatenate([a_ref[0], a_ref[1]], axis=1)
    z = agg * dinv[:, None] + b_ref[...][None, :]
    h = jnp.where(z > 0, z, jnp.exp(jnp.minimum(z, 0.0)) - 1.0)
    o_ref[0] = jnp.dot(h, w_ref[...],
                       preferred_element_type=jnp.float32) * dinv[:, None]


def _tc3_body(a_ref, deg_ref, b_ref, o_ref):
    dinv = _dinv_blk(deg_ref)
    agg = jnp.concatenate([a_ref[0], a_ref[1]], axis=1)
    o_ref[...] = agg * dinv[:, None] + b_ref[...][None, :]


def _tc1(x, W, deg2, N_pad, blk):
    N, Din = x.shape
    Dh = W.shape[1] // NC
    grid = (NC, N // blk)
    return pl.pallas_call(
        _tc1_body,
        grid=grid,
        in_specs=[
            pl.BlockSpec((blk, Din), lambda c, i: (i, 0)),
            pl.BlockSpec((Din, Dh), lambda c, i: (0, c)),
            pl.BlockSpec((blk, NC * NS), lambda c, i: (i, 0)),
        ],
        out_specs=pl.BlockSpec((1, blk, Dh), lambda c, i: (c, i, 0)),
        out_shape=jax.ShapeDtypeStruct((NC, N_pad, Dh), jnp.float32),
    )(x, W, deg2)


def _tc2(agg, deg2, b, W, blk):
    _, N_pad, Dh = agg.shape
    N = deg2.shape[0]
    D = W.shape[0]
    grid = (NC, N // blk)
    return pl.pallas_call(
        _tc2_body,
        grid=grid,
        in_specs=[
            pl.BlockSpec((NC, blk, Dh), lambda c, i: (0, i, 0)),
            pl.BlockSpec((blk, NC * NS), lambda c, i: (i, 0)),
            pl.BlockSpec((D,), lambda c, i: (0,)),
            pl.BlockSpec((D, Dh), lambda c, i: (0, c)),
        ],
        out_specs=pl.BlockSpec((1, blk, Dh), lambda c, i: (c, i, 0)),
        out_shape=jax.ShapeDtypeStruct((NC, N_pad, Dh), jnp.float32),
    )(agg, deg2, b, W)


def _tc3(agg, deg2, b, blk):
    _, N_pad, Dh = agg.shape
    N = deg2.shape[0]
    D = Dh * NC
    grid = (N // blk,)
    return pl.pallas_call(
        _tc3_body,
        grid=grid,
        in_specs=[
            pl.BlockSpec((NC, blk, Dh), lambda i: (0, i, 0)),
            pl.BlockSpec((blk, NC * NS), lambda i: (i, 0)),
            pl.BlockSpec((D,), lambda i: (0,)),
        ],
        out_specs=pl.BlockSpec((blk, D), lambda i: (i, 0)),
        out_shape=jax.ShapeDtypeStruct((N, D), jnp.float32),
    )(agg, deg2, b)


@jax.jit
def kernel(x, edge_index, W1, b1, W2, b2):
    N, Din = x.shape
    E = edge_index.shape[1]
    Dh = W1.shape[1] // NC
    N_pad = ((N + NS * LANES - 1) // (NS * LANES)) * (NS * LANES)
    blk = 1000

    src = edge_index[0]
    dst = edge_index[1]

    gran = NS * 128 * 2
    E_pad = ((E + gran - 1) // gran) * gran
    n_extra = E_pad - E
    pad_idx = N + (jnp.arange(n_extra, dtype=jnp.int32) % (N_pad - N))
    src = jnp.concatenate([src, pad_idx])
    dst = jnp.concatenate([dst, pad_idx])

    sc_agg = _make_sc_agg(E_pad, N_pad, Dh)
    src2d = src.reshape(E_pad // 128, 128)
    dst2d = dst.reshape(E_pad // 128, 128)

    zeros_h = jnp.zeros((N_pad // 128, 128), jnp.float32)
    degp = _make_sc_deg(E_pad, N_pad)(dst, zeros_h)
    deg2 = degp.reshape(NC * NS, N_pad)[:, :N].T

    hp1 = _tc1(x, W1, deg2, N_pad, blk)
    agg1 = sc_agg(hp1, src2d, dst2d)
    hp2 = _tc2(agg1, deg2, b1, W2, blk)
    agg2 = sc_agg(hp2, src2d, dst2d)
    return _tc3(agg2, deg2, b2, blk)


# speedup vs baseline: 18.4334x; 1.0420x over previous
import functools

import jax
import jax.numpy as jnp
from jax import lax
from jax.experimental import pallas as pl
from jax.experimental.pallas import tpu as pltpu
from jax.experimental.pallas import tpu_sc as plsc

NC = 2
NS = 16
LANES = 16


def _make_sc_deg(E_pad, N_pad):
    epw = E_pad // (NC * NS)
    CH = 1280
    assert epw % CH == 0 and CH % LANES == 0
    n_ch = epw // CH
    R = N_pad // 128
    mesh = plsc.VectorSubcoreMesh(core_axis_name="c", subcore_axis_name="s",
                                  num_cores=NC, num_subcores=NS)

    @functools.partial(
        pl.kernel,
        out_type=jax.ShapeDtypeStruct((NC * NS, R, 128), jnp.float32),
        mesh=mesh,
        compiler_params=pltpu.CompilerParams(needs_layout_passes=False),
        scratch_types=[
            pltpu.VMEM((CH,), jnp.int32),
            pltpu.VMEM((R, 128), jnp.float32),
        ],
    )
    def deg_kernel(dst_hbm, zeros_hbm, out_hbm, dst_v, hist):
        c = lax.axis_index("c")
        s = lax.axis_index("s")
        wid = c * NS + s
        pltpu.sync_copy(zeros_hbm, hist)
        ones = jnp.full((LANES,), 1.0, jnp.float32)

        def chunk(k, _):
            base = pl.multiple_of(wid * epw + k * CH, 8)
            pltpu.sync_copy(dst_hbm.at[pl.ds(base, CH)], dst_v)

            def vec(j, _):
                iv = dst_v[pl.ds(j * LANES, LANES)]
                ir = lax.shift_right_logical(iv, 7)
                il = lax.bitwise_and(iv, 127)
                plsc.addupdate_scatter(hist, [ir, il], ones)
                return 0

            lax.fori_loop(0, CH // LANES, vec, 0)
            return 0

        lax.fori_loop(0, n_ch, chunk, 0)
        pltpu.sync_copy(hist, out_hbm.at[wid])

    return deg_kernel


def _make_sc_agg(E_pad, N_pad, Dh):
    ept = E_pad // NS
    K = 128
    B = 16
    assert ept % (K * B) == 0
    n_bulks = ept // (K * B)
    rowb = ept // K
    rpt = N_pad // NS
    assert rpt % 8 == 0
    mesh = plsc.VectorSubcoreMesh(core_axis_name="c", subcore_axis_name="s",
                                  num_cores=NC, num_subcores=NS)

    @functools.partial(
        pl.kernel,
        out_type=jax.ShapeDtypeStruct((NC, N_pad, Dh), jnp.float32),
        mesh=mesh,
        scratch_types=[
            pltpu.VMEM((B, K), jnp.int32),
            pltpu.VMEM((B, K), jnp.int32),
            pltpu.VMEM((K, Dh), jnp.float32),
            pltpu.VMEM((K, Dh), jnp.float32),
            pltpu.SemaphoreType.DMA,
            pltpu.SemaphoreType.DMA,
            pltpu.VMEM_SHARED((N_pad, Dh), jnp.float32),
        ],
    )
    def agg_kernel(hp_hbm, src_hbm, dst_hbm, out_hbm,
                   sidx, didx, rows_v0, rows_v1, sem0, sem1, acc):
        c = lax.axis_index("c")
        s = lax.axis_index("s")
        hp_c = hp_hbm.at[c]
        rows = (rows_v0, rows_v1)
        sems = (sem0, sem1)
        pltpu.sync_copy(hp_c.at[pl.ds(s * rpt, rpt)],
                        acc.at[pl.ds(s * rpt, rpt)])
        plsc.subcore_barrier()

        def start_chunk(j, b):
            pltpu.async_copy(hp_c.at[sidx.at[j]], rows[b], sems[b])

        def finish_chunk(j, b):
            pltpu.make_async_copy(hp_c.at[sidx.at[j]], rows[b],
                                  sems[b]).wait()
            pltpu.sync_copy(rows[b], acc.at[didx.at[j]], add=True)

        def bulk(m, _):
            row0 = pl.multiple_of(s * rowb + m * B, 8)
            pltpu.sync_copy(src_hbm.at[pl.ds(row0, B)], sidx)
            pltpu.sync_copy(dst_hbm.at[pl.ds(row0, B)], didx)
            start_chunk(0, 0)

            def pair(j2, _):
                j = j2 * 2
                start_chunk(j + 1, 1)
                finish_chunk(j, 0)

                @pl.when(j + 2 < B)
                def _():
                    start_chunk(j + 2, 0)

                finish_chunk(j + 1, 1)
                return 0

            lax.fori_loop(0, B // 2, pair, 0)
            return 0

        lax.fori_loop(0, n_bulks, bulk, 0)
        plsc.subcore_barrier()
        pltpu.sync_copy(acc.at[pl.ds(s * rpt, rpt)],
                        out_hbm.at[c].at[pl.ds(s * rpt, rpt)])

    return agg_kernel


def _dinv_blk(deg_ref):
    d = deg_ref[...]
    deg = jnp.sum(d, axis=1) + 1.0
    return lax.rsqrt(deg)


def _tc1_body(x_ref, w_ref, deg_ref, o_ref):
    dinv = _dinv_blk(deg_ref)
    h = jnp.dot(x_ref[...], w_ref[...], preferred_element_type=jnp.float32)
    o_ref[0] = h * dinv[:, None]


def _tc2_body(a_ref, deg_ref, b_ref, w_ref, o_ref):
    dinv = _dinv_blk(deg_ref)
    agg = jnp.concatenate([a_ref[0], a_ref[1]], axis=1)
    z = agg * dinv[:, None] + b_ref[...][None, :]
    h = jnp.where(z > 0, z, jnp.exp(jnp.minimum(z, 0.0)) - 1.0)
    o_ref[0] = jnp.dot(h, w_ref[...],
                       preferred_element_type=jnp.float32) * dinv[:, None]


def _tc3_body(a_ref, deg_ref, b_ref, o_ref):
    dinv = _dinv_blk(deg_ref)
    agg = jnp.concatenate([a_ref[0], a_ref[1]], axis=1)
    o_ref[...] = agg * dinv[:, None] + b_ref[...][None, :]


def _tc1(x, W, deg2, N_pad, blk):
    N, Din = x.shape
    Dh = W.shape[1] // NC
    grid = (NC, N // blk)
    return pl.pallas_call(
        _tc1_body,
        grid=grid,
        in_specs=[
            pl.BlockSpec((blk, Din), lambda c, i: (i, 0)),
            pl.BlockSpec((Din, Dh), lambda c, i: (0, c)),
            pl.BlockSpec((blk, NC * NS), lambda c, i: (i, 0)),
        ],
        out_specs=pl.BlockSpec((1, blk, Dh), lambda c, i: (c, i, 0)),
        out_shape=jax.ShapeDtypeStruct((NC, N_pad, Dh), jnp.float32),
    )(x, W, deg2)


def _tc2(agg, deg2, b, W, blk):
    _, N_pad, Dh = agg.shape
    N = deg2.shape[0]
    D = W.shape[0]
    grid = (NC, N // blk)
    return pl.pallas_call(
        _tc2_body,
        grid=grid,
        in_specs=[
            pl.BlockSpec((NC, blk, Dh), lambda c, i: (0, i, 0)),
            pl.BlockSpec((blk, NC * NS), lambda c, i: (i, 0)),
            pl.BlockSpec((D,), lambda c, i: (0,)),
            pl.BlockSpec((D, Dh), lambda c, i: (0, c)),
        ],
        out_specs=pl.BlockSpec((1, blk, Dh), lambda c, i: (c, i, 0)),
        out_shape=jax.ShapeDtypeStruct((NC, N_pad, Dh), jnp.float32),
    )(agg, deg2, b, W)


def _tc3(agg, deg2, b, blk):
    _, N_pad, Dh = agg.shape
    N = deg2.shape[0]
    D = Dh * NC
    grid = (N // blk,)
    return pl.pallas_call(
        _tc3_body,
        grid=grid,
        in_specs=[
            pl.BlockSpec((NC, blk, Dh), lambda i: (0, i, 0)),
            pl.BlockSpec((blk, NC * NS), lambda i: (i, 0)),
            pl.BlockSpec((D,), lambda i: (0,)),
        ],
        out_specs=pl.BlockSpec((blk, D), lambda i: (i, 0)),
        out_shape=jax.ShapeDtypeStruct((N, D), jnp.float32),
    )(agg, deg2, b)


@jax.jit
def kernel(x, edge_index, W1, b1, W2, b2):
    N, Din = x.shape
    E = edge_index.shape[1]
    Dh = W1.shape[1] // NC
    N_pad = ((N + NS * LANES - 1) // (NS * LANES)) * (NS * LANES)
    blk = 2000

    src = edge_index[0]
    dst = edge_index[1]

    gran = NS * 128 * 2
    E_pad = ((E + gran - 1) // gran) * gran
    n_extra = E_pad - E
    pad_idx = N + (jnp.arange(n_extra, dtype=jnp.int32) % (N_pad - N))
    src = jnp.concatenate([src, pad_idx])
    dst = jnp.concatenate([dst, pad_idx])

    sc_agg = _make_sc_agg(E_pad, N_pad, Dh)
    src2d = src.reshape(E_pad // 128, 128)
    dst2d = dst.reshape(E_pad // 128, 128)

    zeros_h = jnp.zeros((N_pad // 128, 128), jnp.float32)
    degp = _make_sc_deg(E_pad, N_pad)(dst, zeros_h)
    deg2 = degp.reshape(NC * NS, N_pad)[:, :N].T

    hp1 = _tc1(x, W1, deg2, N_pad, blk)
    agg1 = sc_agg(hp1, src2d, dst2d)
    hp2 = _tc2(agg1, deg2, b1, W2, blk)
    agg2 = sc_agg(hp2, src2d, dst2d)
    return _tc3(agg2, deg2, b2, blk)
